# Initial kernel scaffold; baseline (speedup 1.0000x reference)
import jax
import jax.numpy as jnp
from jax.experimental import pallas as pl


def kernel(x, edge_index, edge_attr, ne_W1, ne_b1, ne_W2, ne_b2, ee_W1, ee_b1, ee_W2, ee_b2, gn_W1, gn_b1, gn_W2, gn_b2, ln_gamma, ln_beta, pp_W1, pp_b1, pp_W2, pp_b2, pp_W3, pp_b3):
    raise NotImplementedError("write your pallas kernel here")


# speedup vs baseline: 2.6421x; 2.6421x over previous
import functools

import jax
import jax.numpy as jnp
from jax import lax
from jax.experimental import pallas as pl
from jax.experimental.pallas import tpu as pltpu
from jax.experimental.pallas import tpu_sc as plsc

N_NODES = 10000
N_EDGES = 320000

NC = 2
NS = 16
NW = NC * NS
PER_W = N_EDGES // NW
GCHUNK = 2000
SCHUNK = 1000
ROWS_PER_TILE = N_NODES // NS


def _lrelu(v):
    return jnp.where(v >= 0, v, 0.01 * v)


def _node_pre_body(x_ref, w1_ref, b1_ref, w2_ref, b2_ref, wn_ref, o_ref):
    h = _lrelu(jnp.dot(x_ref[...], w1_ref[...],
                       preferred_element_type=jnp.float32) + b1_ref[...])
    node_enc = _lrelu(jnp.dot(h, w2_ref[...],
                              preferred_element_type=jnp.float32) + b2_ref[...])
    o_ref[...] = jnp.dot(node_enc, wn_ref[...],
                         preferred_element_type=jnp.float32)


def _node_pre(x, ne_W1, ne_b1, ne_W2, ne_b2, gnW1_n):
    return pl.pallas_call(
        _node_pre_body,
        out_shape=jax.ShapeDtypeStruct((N_NODES, 16), jnp.float32),
    )(x, ne_W1, ne_b1.reshape(1, -1), ne_W2, ne_b2.reshape(1, -1), gnW1_n)


def _gather_body(tbl_hbm, src_hbm, g_hbm, idx_v, rows_v, sem):
    wid = lax.axis_index("s") * NC + lax.axis_index("c")
    base = wid * PER_W

    def chunk(i, carry):
        b = base + i * GCHUNK
        pltpu.sync_copy(src_hbm.at[pl.ds(b, GCHUNK)], idx_v)
        pltpu.async_copy(tbl_hbm.at[idx_v], rows_v, sem).wait()
        pltpu.sync_copy(rows_v, g_hbm.at[pl.ds(b, GCHUNK)])
        return carry

    lax.fori_loop(0, PER_W // GCHUNK, chunk, 0)


def _gather(node_pre, src):
    return pl.kernel(
        _gather_body,
        out_type=jax.ShapeDtypeStruct((N_EDGES, 16), jnp.float32),
        mesh=plsc.VectorSubcoreMesh(core_axis_name="c", subcore_axis_name="s"),
        scratch_types=[
            pltpu.VMEM((GCHUNK,), jnp.int32),
            pltpu.VMEM((GCHUNK, 16), jnp.float32),
            pltpu.SemaphoreType.DMA,
        ],
        compiler_params=pltpu.CompilerParams(use_tc_tiling_on_sc=False),
    )(node_pre, src)


def _msg_body(ea_ref, g_ref, w1_ref, b1_ref, w2_ref, b2_ref, we_ref,
              gb1_ref, gw2_ref, gb2_ref, o_ref):
    h = _lrelu(jnp.dot(ea_ref[...], w1_ref[...],
                       preferred_element_type=jnp.float32) + b1_ref[...])
    edge_enc = _lrelu(jnp.dot(h, w2_ref[...],
                              preferred_element_type=jnp.float32) + b2_ref[...])
    e_pre = jnp.dot(edge_enc, we_ref[...],
                    preferred_element_type=jnp.float32) + gb1_ref[...]
    h1 = _lrelu(e_pre + g_ref[...])
    o_ref[...] = _lrelu(jnp.dot(h1, gw2_ref[...],
                                preferred_element_type=jnp.float32) + gb2_ref[...])


def _messages(edge_attr, g, ee_W1, ee_b1, ee_W2, ee_b2, gnW1_e, gn_b1,
              gn_W2, gn_b2):
    BE = 8000
    nblk = N_EDGES // BE
    full = lambda shape: pl.BlockSpec(shape, lambda i: (0, 0))
    return pl.pallas_call(
        _msg_body,
        grid=(nblk,),
        in_specs=[
            pl.BlockSpec((BE, 16), lambda i: (i, 0)),
            pl.BlockSpec((BE, 16), lambda i: (i, 0)),
            full((16, 16)), full((1, 16)), full((16, 32)), full((1, 32)),
            full((32, 16)), full((1, 16)), full((16, 64)), full((1, 64)),
        ],
        out_specs=pl.BlockSpec((BE, 64), lambda i: (i, 0)),
        out_shape=jax.ShapeDtypeStruct((N_EDGES, 64), jnp.float32),
    )(edge_attr, g, ee_W1, ee_b1.reshape(1, -1), ee_W2, ee_b2.reshape(1, -1),
      gnW1_e, gn_b1.reshape(1, -1), gn_W2, gn_b2.reshape(1, -1))


def _scatter_body(msg_hbm, dst_hbm, out_hbm, idx_v, upd_v, acc_sh):
    c = lax.axis_index("c")
    s = lax.axis_index("s")
    wid = s * NC + c

    def zrow(i, carry):
        for j in range(64 // 16):
            upd_v[i, pl.ds(j * 16, 16)] = jnp.zeros((16,), jnp.float32)
        return carry

    lax.fori_loop(0, ROWS_PER_TILE, zrow, 0)
    pltpu.sync_copy(upd_v.at[pl.ds(0, ROWS_PER_TILE)],
                    acc_sh.at[pl.ds(s * ROWS_PER_TILE, ROWS_PER_TILE)])
    plsc.subcore_barrier()

    base = wid * PER_W

    def chunk(i, carry):
        b = base + i * SCHUNK
        pltpu.sync_copy(dst_hbm.at[pl.ds(b, SCHUNK)], idx_v)
        pltpu.sync_copy(msg_hbm.at[pl.ds(b, SCHUNK)], upd_v)
        pltpu.sync_copy(upd_v, acc_sh.at[idx_v], add=True)
        return carry

    lax.fori_loop(0, PER_W // SCHUNK, chunk, 0)
    plsc.subcore_barrier()

    row0 = c * N_NODES + s * ROWS_PER_TILE
    pltpu.sync_copy(acc_sh.at[pl.ds(s * ROWS_PER_TILE, ROWS_PER_TILE)],
                    out_hbm.at[pl.ds(row0, ROWS_PER_TILE)])


def _scatter_add(msg, dst):
    return pl.kernel(
        _scatter_body,
        out_type=jax.ShapeDtypeStruct((NC * N_NODES, 64), jnp.float32),
        mesh=plsc.VectorSubcoreMesh(core_axis_name="c", subcore_axis_name="s"),
        scratch_types=[
            pltpu.VMEM((SCHUNK,), jnp.int32),
            pltpu.VMEM((SCHUNK, 64), jnp.float32),
            pltpu.VMEM_SHARED((N_NODES, 64), jnp.float32),
        ],
        compiler_params=pltpu.CompilerParams(use_tc_tiling_on_sc=False),
    )(msg, dst)


def _post_body(a_ref, b_ref, x_ref, g_ref, bt_ref, w1_ref, b1_ref,
               w2_ref, b2_ref, w3_ref, b3_ref, o_ref):
    gnn = a_ref[...] + b_ref[...]
    pp = jnp.concatenate([gnn, x_ref[...]], axis=1)
    mean = jnp.mean(pp, axis=-1, keepdims=True)
    var = jnp.mean((pp - mean) ** 2, axis=-1, keepdims=True)
    h = (pp - mean) / jnp.sqrt(var + 1e-5) * g_ref[...] + bt_ref[...]
    h = _lrelu(jnp.dot(h, w1_ref[...],
                       preferred_element_type=jnp.float32) + b1_ref[...])
    h = _lrelu(jnp.dot(h, w2_ref[...],
                       preferred_element_type=jnp.float32) + b2_ref[...])
    o_ref[...] = jnp.dot(h, w3_ref[...],
                         preferred_element_type=jnp.float32) + b3_ref[...]


def _post(acc2, x, ln_gamma, ln_beta, pp_W1, pp_b1, pp_W2, pp_b2, pp_W3, pp_b3):
    BN = 1000
    nblk = N_NODES // BN
    full = lambda shape: pl.BlockSpec(shape, lambda i: (0, 0))
    return pl.pallas_call(
        _post_body,
        grid=(nblk,),
        in_specs=[
            pl.BlockSpec((BN, 64), lambda i: (i, 0)),
            pl.BlockSpec((BN, 64), lambda i: (i + N_NODES // BN, 0)),
            pl.BlockSpec((BN, 128), lambda i: (i, 0)),
            full((1, 192)), full((1, 192)),
            full((192, 32)), full((1, 32)), full((32, 32)), full((1, 32)),
            full((32, 128)), full((1, 128)),
        ],
        out_specs=pl.BlockSpec((BN, 128), lambda i: (i, 0)),
        out_shape=jax.ShapeDtypeStruct((N_NODES, 128), jnp.float32),
    )(acc2, acc2, x, ln_gamma.reshape(1, -1), ln_beta.reshape(1, -1),
      pp_W1, pp_b1.reshape(1, -1), pp_W2, pp_b2.reshape(1, -1),
      pp_W3, pp_b3.reshape(1, -1))


def kernel(x, edge_index, edge_attr, ne_W1, ne_b1, ne_W2, ne_b2,
           ee_W1, ee_b1, ee_W2, ee_b2, gn_W1, gn_b1, gn_W2, gn_b2,
           ln_gamma, ln_beta, pp_W1, pp_b1, pp_W2, pp_b2, pp_W3, pp_b3):
    src = edge_index[0].astype(jnp.int32)
    dst = edge_index[1].astype(jnp.int32)
    gnW1_e = gn_W1[:32]
    gnW1_n = gn_W1[32:]

    node_pre = _node_pre(x, ne_W1, ne_b1, ne_W2, ne_b2, gnW1_n)
    g = _gather(node_pre, src)
    msg = _messages(edge_attr, g, ee_W1, ee_b1, ee_W2, ee_b2, gnW1_e,
                    gn_b1, gn_W2, gn_b2)
    acc2 = _scatter_add(msg, dst)
    return _post(acc2, x, ln_gamma, ln_beta,
                 pp_W1, pp_b1, pp_W2, pp_b2, pp_W3, pp_b3)


# speedup vs baseline: 2.9094x; 1.1012x over previous
import jax
import jax.numpy as jnp
from jax import lax
from jax.experimental import pallas as pl
from jax.experimental.pallas import tpu as pltpu
from jax.experimental.pallas import tpu_sc as plsc

N_NODES = 10000
N_EDGES = 320000
EP = N_EDGES // 8
EQ = N_EDGES // 4

NC = 2
NS = 16
NW = NC * NS
PER_W = N_EDGES // NW
GCHUNK = 2000
SCHUNK = 1000
ROWS_PER_TILE = N_NODES // NS


def _lrelu(v):
    return jnp.where(v >= 0, v, 0.01 * v)


def _node_pre_body(x_ref, w1_ref, b1_ref, w2_ref, b2_ref, wn_ref, o_ref):
    h = _lrelu(jnp.dot(x_ref[...], w1_ref[...],
                       preferred_element_type=jnp.float32) + b1_ref[...])
    node_enc = _lrelu(jnp.dot(h, w2_ref[...],
                              preferred_element_type=jnp.float32) + b2_ref[...])
    o_ref[...] = jnp.dot(node_enc, wn_ref[...],
                         preferred_element_type=jnp.float32)


def _node_pre(x, ne_W1, ne_b1, ne_W2, ne_b2, gnW1_n):
    return pl.pallas_call(
        _node_pre_body,
        out_shape=jax.ShapeDtypeStruct((N_NODES, 16), jnp.float32),
    )(x, ne_W1, ne_b1.reshape(1, -1), ne_W2, ne_b2.reshape(1, -1), gnW1_n)


def _gather_body(tbl_hbm, src_hbm, g_hbm, idx_v, rows_v, sem):
    wid = lax.axis_index("s") * NC + lax.axis_index("c")
    base = wid * PER_W

    def chunk(i, carry):
        b = base + i * GCHUNK
        pltpu.sync_copy(src_hbm.at[pl.ds(b, GCHUNK)], idx_v)
        pltpu.async_copy(tbl_hbm.at[idx_v], rows_v, sem).wait()
        pltpu.sync_copy(rows_v, g_hbm.at[pl.ds(b, GCHUNK)])
        return carry

    lax.fori_loop(0, PER_W // GCHUNK, chunk, 0)


def _gather(node_pre, src):
    return pl.kernel(
        _gather_body,
        out_type=jax.ShapeDtypeStruct((N_EDGES, 16), jnp.float32),
        mesh=plsc.VectorSubcoreMesh(core_axis_name="c", subcore_axis_name="s"),
        scratch_types=[
            pltpu.VMEM((GCHUNK,), jnp.int32),
            pltpu.VMEM((GCHUNK, 16), jnp.float32),
            pltpu.SemaphoreType.DMA,
        ],
        compiler_params=pltpu.CompilerParams(use_tc_tiling_on_sc=False),
    )(node_pre, src)


def _msg_body(ea_ref, g_ref, w1_ref, b1_ref, w2_ref, b2_ref, we_ref,
              gb1_ref, gw2_ref, gb2_ref, o0_ref, o1_ref, o2_ref, o3_ref):
    h = _lrelu(jnp.dot(ea_ref[...], w1_ref[...],
                       preferred_element_type=jnp.float32) + b1_ref[...])
    enc = _lrelu(jnp.dot(h, w2_ref[...],
                         preferred_element_type=jnp.float32) + b2_ref[...])
    e_pre = jnp.dot(enc, we_ref[...],
                    preferred_element_type=jnp.float32) + gb1_ref[...]
    h1 = _lrelu(e_pre + g_ref[...])
    msg = _lrelu(jnp.dot(h1, gw2_ref[...],
                         preferred_element_type=jnp.float32) + gb2_ref[...])
    o0_ref[...] = msg[:, 0:128]
    o1_ref[...] = msg[:, 128:256]
    o2_ref[...] = msg[:, 256:384]
    o3_ref[...] = msg[:, 384:512]


def _messages(ea_p, g_p, ee_W1, ee_b1, ee_W2, ee_b2, gnW1_e, gn_b1,
              gn_W2, gn_b2):
    BP = 1000
    nblk = EP // BP
    eye8 = jnp.eye(8, dtype=jnp.float32)
    w1 = jnp.kron(eye8, ee_W1)
    b1 = jnp.tile(ee_b1, 8).reshape(1, -1)
    w2 = jnp.kron(eye8, ee_W2)
    b2 = jnp.tile(ee_b2, 8).reshape(1, -1)
    we = jnp.kron(eye8, gnW1_e)
    gb1 = jnp.tile(gn_b1, 8).reshape(1, -1)
    gw2 = jnp.kron(eye8, gn_W2)
    gb2 = jnp.tile(gn_b2, 8).reshape(1, -1)
    full = lambda shape: pl.BlockSpec(shape, lambda i: (0, 0))
    blk = pl.BlockSpec((BP, 128), lambda i: (i, 0))
    return pl.pallas_call(
        _msg_body,
        grid=(nblk,),
        in_specs=[
            blk, blk,
            full((128, 128)), full((1, 128)), full((128, 256)), full((1, 256)),
            full((256, 128)), full((1, 128)), full((128, 512)), full((1, 512)),
        ],
        out_specs=[blk, blk, blk, blk],
        out_shape=[jax.ShapeDtypeStruct((EP, 128), jnp.float32)] * 4,
    )(ea_p, g_p, w1, b1, w2, b2, we, gb1, gw2, gb2)


def _scatter_body(m0, m1, m2, m3, dst_hbm, out_hbm, idx_v, upd_v, acc_sh):
    c = lax.axis_index("c")
    s = lax.axis_index("s")
    wid = s * NC + c

    def zrow(i, carry):
        for j in range(64 // 16):
            upd_v[i, pl.ds(j * 16, 16)] = jnp.zeros((16,), jnp.float32)
        return carry

    lax.fori_loop(0, ROWS_PER_TILE, zrow, 0)
    pltpu.sync_copy(upd_v.at[pl.ds(0, ROWS_PER_TILE)],
                    acc_sh.at[pl.ds(s * ROWS_PER_TILE, ROWS_PER_TILE)])
    plsc.subcore_barrier()

    base = wid * PER_W
    mbase = (wid % 8) * PER_W
    qi = wid // 8

    for qj, mref in enumerate((m0, m1, m2, m3)):
        @pl.when(qi == qj)
        def _(mref=mref):
            def chunk(i, carry):
                pltpu.sync_copy(dst_hbm.at[pl.ds(base + i * SCHUNK, SCHUNK)],
                                idx_v)
                pltpu.sync_copy(mref.at[pl.ds(mbase + i * SCHUNK, SCHUNK)],
                                upd_v)
                pltpu.sync_copy(upd_v, acc_sh.at[idx_v], add=True)
                return carry

            lax.fori_loop(0, PER_W // SCHUNK, chunk, 0)

    plsc.subcore_barrier()

    row0 = c * N_NODES + s * ROWS_PER_TILE
    pltpu.sync_copy(acc_sh.at[pl.ds(s * ROWS_PER_TILE, ROWS_PER_TILE)],
                    out_hbm.at[pl.ds(row0, ROWS_PER_TILE)])


def _scatter_add(mq, dst_p):
    return pl.kernel(
        _scatter_body,
        out_type=jax.ShapeDtypeStruct((NC * N_NODES, 64), jnp.float32),
        mesh=plsc.VectorSubcoreMesh(core_axis_name="c", subcore_axis_name="s"),
        scratch_types=[
            pltpu.VMEM((SCHUNK,), jnp.int32),
            pltpu.VMEM((SCHUNK, 64), jnp.float32),
            pltpu.VMEM_SHARED((N_NODES, 64), jnp.float32),
        ],
        compiler_params=pltpu.CompilerParams(use_tc_tiling_on_sc=False),
    )(mq[0], mq[1], mq[2], mq[3], dst_p)


def _post_body(a_ref, b_ref, x_ref, g_ref, bt_ref, w1_ref, b1_ref,
               w2_ref, b2_ref, w3_ref, b3_ref, o_ref):
    gnn = a_ref[...] + b_ref[...]
    pp = jnp.concatenate([gnn, x_ref[...]], axis=1)
    mean = jnp.mean(pp, axis=-1, keepdims=True)
    var = jnp.mean((pp - mean) ** 2, axis=-1, keepdims=True)
    h = (pp - mean) / jnp.sqrt(var + 1e-5) * g_ref[...] + bt_ref[...]
    h = _lrelu(jnp.dot(h, w1_ref[...],
                       preferred_element_type=jnp.float32) + b1_ref[...])
    h = _lrelu(jnp.dot(h, w2_ref[...],
                       preferred_element_type=jnp.float32) + b2_ref[...])
    o_ref[...] = jnp.dot(h, w3_ref[...],
                         preferred_element_type=jnp.float32) + b3_ref[...]


def _post(acc2, x, ln_gamma, ln_beta, pp_W1, pp_b1, pp_W2, pp_b2, pp_W3, pp_b3):
    BN = 1000
    nblk = N_NODES // BN
    full = lambda shape: pl.BlockSpec(shape, lambda i: (0, 0))
    return pl.pallas_call(
        _post_body,
        grid=(nblk,),
        in_specs=[
            pl.BlockSpec((BN, 64), lambda i: (i, 0)),
            pl.BlockSpec((BN, 64), lambda i: (i + N_NODES // BN, 0)),
            pl.BlockSpec((BN, 128), lambda i: (i, 0)),
            full((1, 192)), full((1, 192)),
            full((192, 32)), full((1, 32)), full((32, 32)), full((1, 32)),
            full((32, 128)), full((1, 128)),
        ],
        out_specs=pl.BlockSpec((BN, 128), lambda i: (i, 0)),
        out_shape=jax.ShapeDtypeStruct((N_NODES, 128), jnp.float32),
    )(acc2, acc2, x, ln_gamma.reshape(1, -1), ln_beta.reshape(1, -1),
      pp_W1, pp_b1.reshape(1, -1), pp_W2, pp_b2.reshape(1, -1),
      pp_W3, pp_b3.reshape(1, -1))


def kernel(x, edge_index, edge_attr, ne_W1, ne_b1, ne_W2, ne_b2,
           ee_W1, ee_b1, ee_W2, ee_b2, gn_W1, gn_b1, gn_W2, gn_b2,
           ln_gamma, ln_beta, pp_W1, pp_b1, pp_W2, pp_b2, pp_W3, pp_b3):
    src = edge_index[0].astype(jnp.int32)
    dst = edge_index[1].astype(jnp.int32)
    gnW1_e = gn_W1[:32]
    gnW1_n = gn_W1[32:]

    node_pre = _node_pre(x, ne_W1, ne_b1, ne_W2, ne_b2, gnW1_n)
    g = _gather(node_pre, src)
    g_p = jnp.reshape(g, (EP, 128))
    ea_p = jnp.reshape(edge_attr, (EP, 128))
    mq = _messages(ea_p, g_p, ee_W1, ee_b1, ee_W2, ee_b2, gnW1_e,
                   gn_b1, gn_W2, gn_b2)
    mq = [jnp.reshape(m, (EQ, 64)) for m in mq]
    dst_p = dst.reshape(EP, 4, 2).transpose(1, 0, 2).reshape(-1)
    acc2 = _scatter_add(mq, dst_p)
    return _post(acc2, x, ln_gamma, ln_beta,
                 pp_W1, pp_b1, pp_W2, pp_b2, pp_W3, pp_b3)


# speedup vs baseline: 4.3730x; 1.5031x over previous
import jax
import jax.numpy as jnp
from jax import lax
from jax.experimental import pallas as pl
from jax.experimental.pallas import tpu as pltpu
from jax.experimental.pallas import tpu_sc as plsc

N_NODES = 10000
N_EDGES = 320000
EP = N_EDGES // 8
EQ = N_EDGES // 4

NC = 2
NS = 16
NW = NC * NS
PER_W = N_EDGES // NW
GCHUNK = 2000
SCHUNK = 400
ROWS_PER_TILE = N_NODES // NS


def _lrelu(v):
    return jnp.where(v >= 0, v, 0.01 * v)


def _node_pre_body(x_ref, w1_ref, b1_ref, w2_ref, b2_ref, wn_ref, o_ref):
    h = _lrelu(jnp.dot(x_ref[...], w1_ref[...],
                       preferred_element_type=jnp.float32) + b1_ref[...])
    node_enc = _lrelu(jnp.dot(h, w2_ref[...],
                              preferred_element_type=jnp.float32) + b2_ref[...])
    o_ref[...] = jnp.dot(node_enc, wn_ref[...],
                         preferred_element_type=jnp.float32)


def _node_pre(x, ne_W1, ne_b1, ne_W2, ne_b2, gnW1_n):
    return pl.pallas_call(
        _node_pre_body,
        out_shape=jax.ShapeDtypeStruct((N_NODES, 16), jnp.float32),
    )(x, ne_W1, ne_b1.reshape(1, -1), ne_W2, ne_b2.reshape(1, -1), gnW1_n)


def _gather_body(tbl_hbm, src_hbm, g_hbm, idx_v, rows_v, sem):
    wid = lax.axis_index("s") * NC + lax.axis_index("c")
    base = wid * PER_W

    def chunk(i, carry):
        b = base + i * GCHUNK
        pltpu.sync_copy(src_hbm.at[pl.ds(b, GCHUNK)], idx_v)
        pltpu.async_copy(tbl_hbm.at[idx_v], rows_v, sem).wait()
        pltpu.sync_copy(rows_v, g_hbm.at[pl.ds(b, GCHUNK)])
        return carry

    lax.fori_loop(0, PER_W // GCHUNK, chunk, 0)


def _gather(node_pre, src):
    return pl.kernel(
        _gather_body,
        out_type=jax.ShapeDtypeStruct((N_EDGES, 16), jnp.float32),
        mesh=plsc.VectorSubcoreMesh(core_axis_name="c", subcore_axis_name="s"),
        scratch_types=[
            pltpu.VMEM((GCHUNK,), jnp.int32),
            pltpu.VMEM((GCHUNK, 16), jnp.float32),
            pltpu.SemaphoreType.DMA,
        ],
        compiler_params=pltpu.CompilerParams(use_tc_tiling_on_sc=False),
    )(node_pre, src)


def _msg_body(ea_ref, g_ref, w1_ref, b1_ref, w2_ref, b2_ref, we_ref,
              gb1_ref, gw2_ref, gb2_ref, o0_ref, o1_ref, o2_ref, o3_ref):
    h = _lrelu(jnp.dot(ea_ref[...], w1_ref[...],
                       preferred_element_type=jnp.float32) + b1_ref[...])
    enc = _lrelu(jnp.dot(h, w2_ref[...],
                         preferred_element_type=jnp.float32) + b2_ref[...])
    e_pre = jnp.dot(enc, we_ref[...],
                    preferred_element_type=jnp.float32) + gb1_ref[...]
    h1 = _lrelu(e_pre + g_ref[...])
    msg = _lrelu(jnp.dot(h1, gw2_ref[...],
                         preferred_element_type=jnp.float32) + gb2_ref[...])
    o0_ref[...] = msg[:, 0:128]
    o1_ref[...] = msg[:, 128:256]
    o2_ref[...] = msg[:, 256:384]
    o3_ref[...] = msg[:, 384:512]


def _messages(ea_p, g_p, ee_W1, ee_b1, ee_W2, ee_b2, gnW1_e, gn_b1,
              gn_W2, gn_b2):
    BP = 1000
    nblk = EP // BP
    eye8 = jnp.eye(8, dtype=jnp.float32)
    w1 = jnp.kron(eye8, ee_W1)
    b1 = jnp.tile(ee_b1, 8).reshape(1, -1)
    w2 = jnp.kron(eye8, ee_W2)
    b2 = jnp.tile(ee_b2, 8).reshape(1, -1)
    we = jnp.kron(eye8, gnW1_e)
    gb1 = jnp.tile(gn_b1, 8).reshape(1, -1)
    gw2 = jnp.kron(eye8, gn_W2)
    gb2 = jnp.tile(gn_b2, 8).reshape(1, -1)
    full = lambda shape: pl.BlockSpec(shape, lambda i: (0, 0))
    blk = pl.BlockSpec((BP, 128), lambda i: (i, 0))
    return pl.pallas_call(
        _msg_body,
        grid=(nblk,),
        in_specs=[
            blk, blk,
            full((128, 128)), full((1, 128)), full((128, 256)), full((1, 256)),
            full((256, 128)), full((1, 128)), full((128, 512)), full((1, 512)),
        ],
        out_specs=[blk, blk, blk, blk],
        out_shape=[jax.ShapeDtypeStruct((EP, 128), jnp.float32)] * 4,
    )(ea_p, g_p, w1, b1, w2, b2, we, gb1, gw2, gb2)


def _scatter_body(m0, m1, m2, m3, dst_hbm, pos_hbm, out_hbm, pos_v, idx_v,
                  upd_v, acc_sh):
    c = lax.axis_index("c")
    s = lax.axis_index("s")
    wid = s * NC + c
    qi = wid // 8
    sub = wid % 8

    def zrow(i, carry):
        for j in range(64 // 16):
            upd_v[i, pl.ds(j * 16, 16)] = jnp.zeros((16,), jnp.float32)
        return carry

    lax.fori_loop(0, SCHUNK, zrow, 0)
    pltpu.sync_copy(upd_v.at[pl.ds(0, SCHUNK)],
                    acc_sh.at[pl.ds(s * ROWS_PER_TILE, SCHUNK)])
    pltpu.sync_copy(upd_v.at[pl.ds(0, ROWS_PER_TILE - SCHUNK)],
                    acc_sh.at[pl.ds(s * ROWS_PER_TILE + SCHUNK,
                                    ROWS_PER_TILE - SCHUNK)])
    plsc.subcore_barrier()

    nchunk = PER_W // SCHUNK

    for qj, mref in enumerate((m0, m1, m2, m3)):
        @pl.when(qi == qj)
        def _(mref=mref):
            def chunk(i, carry):
                base = wid * PER_W + i * SCHUNK
                pltpu.sync_copy(pos_hbm.at[pl.ds(base, SCHUNK)], pos_v)
                pltpu.sync_copy(dst_hbm.at[pos_v], idx_v)
                pltpu.sync_copy(mref.at[pl.ds(sub * PER_W + i * SCHUNK,
                                              SCHUNK)], upd_v)
                pltpu.sync_copy(upd_v, acc_sh.at[idx_v], add=True)
                return carry

            lax.fori_loop(0, nchunk, chunk, 0)

    plsc.subcore_barrier()

    row0 = c * N_NODES + s * ROWS_PER_TILE
    pltpu.sync_copy(acc_sh.at[pl.ds(s * ROWS_PER_TILE, ROWS_PER_TILE)],
                    out_hbm.at[pl.ds(row0, ROWS_PER_TILE)])


def _scatter_add(mq, dst, pos):
    return pl.kernel(
        _scatter_body,
        out_type=jax.ShapeDtypeStruct((NC * N_NODES, 64), jnp.float32),
        mesh=plsc.VectorSubcoreMesh(core_axis_name="c", subcore_axis_name="s"),
        scratch_types=[
            pltpu.VMEM((SCHUNK,), jnp.int32),
            pltpu.VMEM((SCHUNK,), jnp.int32),
            pltpu.VMEM((SCHUNK, 64), jnp.float32),
            pltpu.VMEM_SHARED((N_NODES, 64), jnp.float32),
        ],
        compiler_params=pltpu.CompilerParams(use_tc_tiling_on_sc=False),
    )(mq[0], mq[1], mq[2], mq[3], dst, pos)


def _post_body(a_ref, b_ref, x_ref, g_ref, bt_ref, w1_ref, b1_ref,
               w2_ref, b2_ref, w3_ref, b3_ref, o_ref):
    gnn = a_ref[...] + b_ref[...]
    pp = jnp.concatenate([gnn, x_ref[...]], axis=1)
    mean = jnp.mean(pp, axis=-1, keepdims=True)
    var = jnp.mean((pp - mean) ** 2, axis=-1, keepdims=True)
    h = (pp - mean) / jnp.sqrt(var + 1e-5) * g_ref[...] + bt_ref[...]
    h = _lrelu(jnp.dot(h, w1_ref[...],
                       preferred_element_type=jnp.float32) + b1_ref[...])
    h = _lrelu(jnp.dot(h, w2_ref[...],
                       preferred_element_type=jnp.float32) + b2_ref[...])
    o_ref[...] = jnp.dot(h, w3_ref[...],
                         preferred_element_type=jnp.float32) + b3_ref[...]


def _post(acc2, x, ln_gamma, ln_beta, pp_W1, pp_b1, pp_W2, pp_b2, pp_W3, pp_b3):
    BN = 1000
    nblk = N_NODES // BN
    full = lambda shape: pl.BlockSpec(shape, lambda i: (0, 0))
    return pl.pallas_call(
        _post_body,
        grid=(nblk,),
        in_specs=[
            pl.BlockSpec((BN, 64), lambda i: (i, 0)),
            pl.BlockSpec((BN, 64), lambda i: (i + N_NODES // BN, 0)),
            pl.BlockSpec((BN, 128), lambda i: (i, 0)),
            full((1, 192)), full((1, 192)),
            full((192, 32)), full((1, 32)), full((32, 32)), full((1, 32)),
            full((32, 128)), full((1, 128)),
        ],
        out_specs=pl.BlockSpec((BN, 128), lambda i: (i, 0)),
        out_shape=jax.ShapeDtypeStruct((N_NODES, 128), jnp.float32),
    )(acc2, acc2, x, ln_gamma.reshape(1, -1), ln_beta.reshape(1, -1),
      pp_W1, pp_b1.reshape(1, -1), pp_W2, pp_b2.reshape(1, -1),
      pp_W3, pp_b3.reshape(1, -1))


def kernel(x, edge_index, edge_attr, ne_W1, ne_b1, ne_W2, ne_b2,
           ee_W1, ee_b1, ee_W2, ee_b2, gn_W1, gn_b1, gn_W2, gn_b2,
           ln_gamma, ln_beta, pp_W1, pp_b1, pp_W2, pp_b2, pp_W3, pp_b3):
    src = edge_index[0].astype(jnp.int32)
    dst = edge_index[1].astype(jnp.int32)
    gnW1_e = gn_W1[:32]
    gnW1_n = gn_W1[32:]

    node_pre = _node_pre(x, ne_W1, ne_b1, ne_W2, ne_b2, gnW1_n)
    g = _gather(node_pre, src)
    g_p = jnp.reshape(g, (EP, 128))
    ea_p = jnp.reshape(edge_attr, (EP, 128))
    mq = _messages(ea_p, g_p, ee_W1, ee_b1, ee_W2, ee_b2, gnW1_e,
                   gn_b1, gn_W2, gn_b2)
    mq = [jnp.reshape(m, (EQ, 64)) for m in mq]
    q = jnp.arange(EQ, dtype=jnp.int32)
    qbase = 8 * (q // 2) + q % 2
    pos = jnp.concatenate([qbase + 2 * j for j in range(4)])
    acc2 = _scatter_add(mq, dst, pos)
    return _post(acc2, x, ln_gamma, ln_beta,
                 pp_W1, pp_b1, pp_W2, pp_b2, pp_W3, pp_b3)


# speedup vs baseline: 4.8037x; 1.0985x over previous
import jax
import jax.numpy as jnp
from jax import lax
from jax.experimental import pallas as pl
from jax.experimental.pallas import tpu as pltpu
from jax.experimental.pallas import tpu_sc as plsc

N_NODES = 10000
N_EDGES = 320000
EP = N_EDGES // 8
EQ = N_EDGES // 4

NC = 2
NS = 16
NW = NC * NS
PER_W = N_EDGES // NW
GCHUNK = 2000
SCHUNK = 1000
ROWS_PER_TILE = N_NODES // NS


def _lrelu(v):
    return jnp.where(v >= 0, v, 0.01 * v)


def _node_pre_body(x_ref, w1_ref, b1_ref, w2_ref, b2_ref, wn_ref, o_ref):
    h = _lrelu(jnp.dot(x_ref[...], w1_ref[...],
                       preferred_element_type=jnp.float32) + b1_ref[...])
    node_enc = _lrelu(jnp.dot(h, w2_ref[...],
                              preferred_element_type=jnp.float32) + b2_ref[...])
    o_ref[...] = jnp.dot(node_enc, wn_ref[...],
                         preferred_element_type=jnp.float32)


def _node_pre(x, ne_W1, ne_b1, ne_W2, ne_b2, gnW1_n):
    return pl.pallas_call(
        _node_pre_body,
        out_shape=jax.ShapeDtypeStruct((N_NODES, 16), jnp.float32),
    )(x, ne_W1, ne_b1.reshape(1, -1), ne_W2, ne_b2.reshape(1, -1), gnW1_n)


def _gather_body(tbl_hbm, src_hbm, g_hbm, idx0, idx1, rows0, rows1,
                 semg0, semg1, semw0, semw1):
    wid = lax.axis_index("s") * NC + lax.axis_index("c")
    base = wid * PER_W
    idxs, rows = (idx0, idx1), (rows0, rows1)
    semg, semw = (semg0, semg1), (semw0, semw1)
    nch = PER_W // GCHUNK

    pltpu.sync_copy(src_hbm.at[pl.ds(base, GCHUNK)], idx0)
    gcp = {0: pltpu.async_copy(tbl_hbm.at[idx0], rows0, semg0)}
    wcp = {}
    for i in range(nch):
        cur, nxt = i % 2, (i + 1) % 2
        if i + 1 < nch:
            if i - 1 >= 0:
                wcp[nxt].wait()
            b1 = base + (i + 1) * GCHUNK
            pltpu.sync_copy(src_hbm.at[pl.ds(b1, GCHUNK)], idxs[nxt])
            gcp[nxt] = pltpu.async_copy(tbl_hbm.at[idxs[nxt]], rows[nxt],
                                        semg[nxt])
        gcp[cur].wait()
        wcp[cur] = pltpu.async_copy(rows[cur],
                                    g_hbm.at[pl.ds(base + i * GCHUNK,
                                                   GCHUNK)], semw[cur])
    wcp[(nch - 2) % 2].wait()
    wcp[(nch - 1) % 2].wait()


def _gather(node_pre, src):
    return pl.kernel(
        _gather_body,
        out_type=jax.ShapeDtypeStruct((N_EDGES, 16), jnp.float32),
        mesh=plsc.VectorSubcoreMesh(core_axis_name="c", subcore_axis_name="s"),
        scratch_types=[
            pltpu.VMEM((GCHUNK,), jnp.int32),
            pltpu.VMEM((GCHUNK,), jnp.int32),
            pltpu.VMEM((GCHUNK, 16), jnp.float32),
            pltpu.VMEM((GCHUNK, 16), jnp.float32),
            pltpu.SemaphoreType.DMA,
            pltpu.SemaphoreType.DMA,
            pltpu.SemaphoreType.DMA,
            pltpu.SemaphoreType.DMA,
        ],
        compiler_params=pltpu.CompilerParams(use_tc_tiling_on_sc=False),
    )(node_pre, src)


def _msg_body(ea_ref, g_ref, w1_ref, b1_ref, w2_ref, b2_ref, we_ref,
              gb1_ref, gw2_ref, gb2_ref, o0_ref, o1_ref, o2_ref, o3_ref):
    h = _lrelu(jnp.dot(ea_ref[...], w1_ref[...],
                       preferred_element_type=jnp.float32) + b1_ref[...])
    enc = _lrelu(jnp.dot(h, w2_ref[...],
                         preferred_element_type=jnp.float32) + b2_ref[...])
    e_pre = jnp.dot(enc, we_ref[...],
                    preferred_element_type=jnp.float32) + gb1_ref[...]
    h1 = _lrelu(e_pre + g_ref[...])
    msg = _lrelu(jnp.dot(h1, gw2_ref[...],
                         preferred_element_type=jnp.float32) + gb2_ref[...])
    o0_ref[...] = msg[:, 0:128]
    o1_ref[...] = msg[:, 128:256]
    o2_ref[...] = msg[:, 256:384]
    o3_ref[...] = msg[:, 384:512]


def _messages(ea_p, g_p, ee_W1, ee_b1, ee_W2, ee_b2, gnW1_e, gn_b1,
              gn_W2, gn_b2):
    BP = 2000
    nblk = EP // BP
    eye8 = jnp.eye(8, dtype=jnp.float32)
    w1 = jnp.kron(eye8, ee_W1)
    b1 = jnp.tile(ee_b1, 8).reshape(1, -1)
    w2 = jnp.kron(eye8, ee_W2)
    b2 = jnp.tile(ee_b2, 8).reshape(1, -1)
    we = jnp.kron(eye8, gnW1_e)
    gb1 = jnp.tile(gn_b1, 8).reshape(1, -1)
    gw2 = jnp.kron(eye8, gn_W2)
    gb2 = jnp.tile(gn_b2, 8).reshape(1, -1)
    full = lambda shape: pl.BlockSpec(shape, lambda i: (0, 0))
    blk = pl.BlockSpec((BP, 128), lambda i: (i, 0))
    return pl.pallas_call(
        _msg_body,
        grid=(nblk,),
        in_specs=[
            blk, blk,
            full((128, 128)), full((1, 128)), full((128, 256)), full((1, 256)),
            full((256, 128)), full((1, 128)), full((128, 512)), full((1, 512)),
        ],
        out_specs=[blk, blk, blk, blk],
        out_shape=[jax.ShapeDtypeStruct((EP, 128), jnp.float32)] * 4,
    )(ea_p, g_p, w1, b1, w2, b2, we, gb1, gw2, gb2)


def _scatter_body(m0, m1, m2, m3, dst_hbm, pos_hbm, out_hbm, pos_v, idx_v,
                  upd_v, acc_sh):
    c = lax.axis_index("c")
    s = lax.axis_index("s")
    wid = s * NC + c
    qi = wid // 8
    sub = wid % 8

    def zrow(i, carry):
        for j in range(64 // 16):
            upd_v[i, pl.ds(j * 16, 16)] = jnp.zeros((16,), jnp.float32)
        return carry

    lax.fori_loop(0, ROWS_PER_TILE, zrow, 0)
    pltpu.sync_copy(upd_v.at[pl.ds(0, ROWS_PER_TILE)],
                    acc_sh.at[pl.ds(s * ROWS_PER_TILE, ROWS_PER_TILE)])
    plsc.subcore_barrier()

    nchunk = PER_W // SCHUNK

    for qj, mref in enumerate((m0, m1, m2, m3)):
        @pl.when(qi == qj)
        def _(mref=mref):
            def chunk(i, carry):
                base = wid * PER_W + i * SCHUNK
                pltpu.sync_copy(pos_hbm.at[pl.ds(base, SCHUNK)], pos_v)
                pltpu.sync_copy(dst_hbm.at[pos_v], idx_v)
                pltpu.sync_copy(mref.at[pl.ds(sub * PER_W + i * SCHUNK,
                                              SCHUNK)], upd_v)
                pltpu.sync_copy(upd_v, acc_sh.at[idx_v], add=True)
                return carry

            lax.fori_loop(0, nchunk, chunk, 0)

    plsc.subcore_barrier()

    row0 = c * N_NODES + s * ROWS_PER_TILE
    pltpu.sync_copy(acc_sh.at[pl.ds(s * ROWS_PER_TILE, ROWS_PER_TILE)],
                    out_hbm.at[pl.ds(row0, ROWS_PER_TILE)])


def _scatter_add(mq, dst, pos):
    return pl.kernel(
        _scatter_body,
        out_type=jax.ShapeDtypeStruct((NC * N_NODES, 64), jnp.float32),
        mesh=plsc.VectorSubcoreMesh(core_axis_name="c", subcore_axis_name="s"),
        scratch_types=[
            pltpu.VMEM((SCHUNK,), jnp.int32),
            pltpu.VMEM((SCHUNK,), jnp.int32),
            pltpu.VMEM((SCHUNK, 64), jnp.float32),
            pltpu.VMEM_SHARED((N_NODES, 64), jnp.float32),
        ],
        compiler_params=pltpu.CompilerParams(use_tc_tiling_on_sc=False),
    )(mq[0], mq[1], mq[2], mq[3], dst, pos)


def _post_body(a_ref, b_ref, x_ref, g_ref, bt_ref, w1_ref, b1_ref,
               w2_ref, b2_ref, w3_ref, b3_ref, o_ref):
    gnn = a_ref[...] + b_ref[...]
    pp = jnp.concatenate([gnn, x_ref[...]], axis=1)
    mean = jnp.mean(pp, axis=-1, keepdims=True)
    var = jnp.mean((pp - mean) ** 2, axis=-1, keepdims=True)
    h = (pp - mean) / jnp.sqrt(var + 1e-5) * g_ref[...] + bt_ref[...]
    h = _lrelu(jnp.dot(h, w1_ref[...],
                       preferred_element_type=jnp.float32) + b1_ref[...])
    h = _lrelu(jnp.dot(h, w2_ref[...],
                       preferred_element_type=jnp.float32) + b2_ref[...])
    o_ref[...] = jnp.dot(h, w3_ref[...],
                         preferred_element_type=jnp.float32) + b3_ref[...]


def _post(acc2, x, ln_gamma, ln_beta, pp_W1, pp_b1, pp_W2, pp_b2, pp_W3, pp_b3):
    BN = 1000
    nblk = N_NODES // BN
    full = lambda shape: pl.BlockSpec(shape, lambda i: (0, 0))
    return pl.pallas_call(
        _post_body,
        grid=(nblk,),
        in_specs=[
            pl.BlockSpec((BN, 64), lambda i: (i, 0)),
            pl.BlockSpec((BN, 64), lambda i: (i + N_NODES // BN, 0)),
            pl.BlockSpec((BN, 128), lambda i: (i, 0)),
            full((1, 192)), full((1, 192)),
            full((192, 32)), full((1, 32)), full((32, 32)), full((1, 32)),
            full((32, 128)), full((1, 128)),
        ],
        out_specs=pl.BlockSpec((BN, 128), lambda i: (i, 0)),
        out_shape=jax.ShapeDtypeStruct((N_NODES, 128), jnp.float32),
    )(acc2, acc2, x, ln_gamma.reshape(1, -1), ln_beta.reshape(1, -1),
      pp_W1, pp_b1.reshape(1, -1), pp_W2, pp_b2.reshape(1, -1),
      pp_W3, pp_b3.reshape(1, -1))


def kernel(x, edge_index, edge_attr, ne_W1, ne_b1, ne_W2, ne_b2,
           ee_W1, ee_b1, ee_W2, ee_b2, gn_W1, gn_b1, gn_W2, gn_b2,
           ln_gamma, ln_beta, pp_W1, pp_b1, pp_W2, pp_b2, pp_W3, pp_b3):
    src = edge_index[0].astype(jnp.int32)
    dst = edge_index[1].astype(jnp.int32)
    gnW1_e = gn_W1[:32]
    gnW1_n = gn_W1[32:]

    node_pre = _node_pre(x, ne_W1, ne_b1, ne_W2, ne_b2, gnW1_n)
    g = _gather(node_pre, src)
    g_p = jnp.reshape(g, (EP, 128))
    ea_p = jnp.reshape(edge_attr, (EP, 128))
    mq = _messages(ea_p, g_p, ee_W1, ee_b1, ee_W2, ee_b2, gnW1_e,
                   gn_b1, gn_W2, gn_b2)
    mq = [jnp.reshape(m, (EQ, 64)) for m in mq]
    q = jnp.arange(EQ, dtype=jnp.int32)
    qbase = 8 * (q // 2) + q % 2
    pos = jnp.concatenate([qbase + 2 * j for j in range(4)])
    acc2 = _scatter_add(mq, dst, pos)
    return _post(acc2, x, ln_gamma, ln_beta,
                 pp_W1, pp_b1, pp_W2, pp_b2, pp_W3, pp_b3)


# speedup vs baseline: 5.3508x; 1.1139x over previous
import jax
import jax.numpy as jnp
from jax import lax
from jax.experimental import pallas as pl
from jax.experimental.pallas import tpu as pltpu
from jax.experimental.pallas import tpu_sc as plsc

N_NODES = 10000
N_EDGES = 320000
EP = N_EDGES // 8
EQ = N_EDGES // 4

NC = 2
NS = 16
NW = NC * NS
PER_W = N_EDGES // NW
GCHUNK = 2000
SCHUNK = 400
ROWS_PER_TILE = N_NODES // NS


def _lrelu(v):
    return jnp.where(v >= 0, v, 0.01 * v)


def _node_pre_body(x_ref, w1_ref, b1_ref, w2_ref, b2_ref, wn_ref, o_ref):
    h = _lrelu(jnp.dot(x_ref[...], w1_ref[...],
                       preferred_element_type=jnp.float32) + b1_ref[...])
    node_enc = _lrelu(jnp.dot(h, w2_ref[...],
                              preferred_element_type=jnp.float32) + b2_ref[...])
    o_ref[...] = jnp.dot(node_enc, wn_ref[...],
                         preferred_element_type=jnp.float32)


def _node_pre(x, ne_W1, ne_b1, ne_W2, ne_b2, gnW1_n):
    return pl.pallas_call(
        _node_pre_body,
        out_shape=jax.ShapeDtypeStruct((N_NODES, 16), jnp.float32),
    )(x, ne_W1, ne_b1.reshape(1, -1), ne_W2, ne_b2.reshape(1, -1), gnW1_n)


def _gather_body(tbl_hbm, src_hbm, g_hbm, idx0, idx1, rows0, rows1,
                 semg0, semg1, semw0, semw1):
    wid = lax.axis_index("s") * NC + lax.axis_index("c")
    base = wid * PER_W
    idxs, rows = (idx0, idx1), (rows0, rows1)
    semg, semw = (semg0, semg1), (semw0, semw1)
    nch = PER_W // GCHUNK

    pltpu.sync_copy(src_hbm.at[pl.ds(base, GCHUNK)], idx0)
    gcp = {0: pltpu.async_copy(tbl_hbm.at[idx0], rows0, semg0)}
    wcp = {}
    for i in range(nch):
        cur, nxt = i % 2, (i + 1) % 2
        if i + 1 < nch:
            if i - 1 >= 0:
                wcp[nxt].wait()
            b1 = base + (i + 1) * GCHUNK
            pltpu.sync_copy(src_hbm.at[pl.ds(b1, GCHUNK)], idxs[nxt])
            gcp[nxt] = pltpu.async_copy(tbl_hbm.at[idxs[nxt]], rows[nxt],
                                        semg[nxt])
        gcp[cur].wait()
        wcp[cur] = pltpu.async_copy(rows[cur],
                                    g_hbm.at[pl.ds(base + i * GCHUNK,
                                                   GCHUNK)], semw[cur])
    wcp[(nch - 2) % 2].wait()
    wcp[(nch - 1) % 2].wait()


def _gather(node_pre, src):
    return pl.kernel(
        _gather_body,
        out_type=jax.ShapeDtypeStruct((N_EDGES, 16), jnp.float32),
        mesh=plsc.VectorSubcoreMesh(core_axis_name="c", subcore_axis_name="s"),
        scratch_types=[
            pltpu.VMEM((GCHUNK,), jnp.int32),
            pltpu.VMEM((GCHUNK,), jnp.int32),
            pltpu.VMEM((GCHUNK, 16), jnp.float32),
            pltpu.VMEM((GCHUNK, 16), jnp.float32),
            pltpu.SemaphoreType.DMA,
            pltpu.SemaphoreType.DMA,
            pltpu.SemaphoreType.DMA,
            pltpu.SemaphoreType.DMA,
        ],
        compiler_params=pltpu.CompilerParams(use_tc_tiling_on_sc=False),
    )(node_pre, src)


def _msg_body(ea_ref, g_ref, w1_ref, b1_ref, w2_ref, b2_ref, we_ref,
              gb1_ref, gw2_ref, gb2_ref, o0_ref, o1_ref, o2_ref, o3_ref):
    h = _lrelu(jnp.dot(ea_ref[...], w1_ref[...],
                       preferred_element_type=jnp.float32) + b1_ref[...])
    enc = _lrelu(jnp.dot(h, w2_ref[...],
                         preferred_element_type=jnp.float32) + b2_ref[...])
    e_pre = jnp.dot(enc, we_ref[...],
                    preferred_element_type=jnp.float32) + gb1_ref[...]
    h1 = _lrelu(e_pre + g_ref[...])
    msg = _lrelu(jnp.dot(h1, gw2_ref[...],
                         preferred_element_type=jnp.float32) + gb2_ref[...])
    o0_ref[...] = msg[:, 0:128]
    o1_ref[...] = msg[:, 128:256]
    o2_ref[...] = msg[:, 256:384]
    o3_ref[...] = msg[:, 384:512]


def _messages(ea_p, g_p, ee_W1, ee_b1, ee_W2, ee_b2, gnW1_e, gn_b1,
              gn_W2, gn_b2):
    BP = 2000
    nblk = EP // BP
    eye8 = jnp.eye(8, dtype=jnp.float32)
    w1 = jnp.kron(eye8, ee_W1)
    b1 = jnp.tile(ee_b1, 8).reshape(1, -1)
    w2 = jnp.kron(eye8, ee_W2)
    b2 = jnp.tile(ee_b2, 8).reshape(1, -1)
    we = jnp.kron(eye8, gnW1_e)
    gb1 = jnp.tile(gn_b1, 8).reshape(1, -1)
    gw2 = jnp.kron(eye8, gn_W2)
    gb2 = jnp.tile(gn_b2, 8).reshape(1, -1)
    full = lambda shape: pl.BlockSpec(shape, lambda i: (0, 0))
    blk = pl.BlockSpec((BP, 128), lambda i: (i, 0))
    return pl.pallas_call(
        _msg_body,
        grid=(nblk,),
        in_specs=[
            blk, blk,
            full((128, 128)), full((1, 128)), full((128, 256)), full((1, 256)),
            full((256, 128)), full((1, 128)), full((128, 512)), full((1, 512)),
        ],
        out_specs=[blk, blk, blk, blk],
        out_shape=[jax.ShapeDtypeStruct((EP, 128), jnp.float32)] * 4,
    )(ea_p, g_p, w1, b1, w2, b2, we, gb1, gw2, gb2)


def _scatter_body(m0, m1, m2, m3, dst_hbm, pos_hbm, out_hbm, pos0, pos1,
                  idx0, idx1, upd0, upd1, semi0, semi1, semm0, semm1, acc_sh):
    c = lax.axis_index("c")
    s = lax.axis_index("s")
    wid = s * NC + c
    qi = wid // 8
    sub = wid % 8

    def zrow(i, carry):
        for j in range(64 // 16):
            upd0[i, pl.ds(j * 16, 16)] = jnp.zeros((16,), jnp.float32)
        return carry

    lax.fori_loop(0, SCHUNK, zrow, 0)
    pltpu.sync_copy(upd0, acc_sh.at[pl.ds(s * ROWS_PER_TILE, SCHUNK)])
    pltpu.sync_copy(upd0.at[pl.ds(0, ROWS_PER_TILE - SCHUNK)],
                    acc_sh.at[pl.ds(s * ROWS_PER_TILE + SCHUNK,
                                    ROWS_PER_TILE - SCHUNK)])
    plsc.subcore_barrier()

    nchunk = PER_W // SCHUNK
    poss, idxs, upds = (pos0, pos1), (idx0, idx1), (upd0, upd1)
    semi, semm = (semi0, semi1), (semm0, semm1)

    for qj, mref in enumerate((m0, m1, m2, m3)):
        @pl.when(qi == qj)
        def _(mref=mref):
            def issue(i, b):
                base = wid * PER_W + i * SCHUNK
                mcp = pltpu.async_copy(
                    mref.at[pl.ds(sub * PER_W + i * SCHUNK, SCHUNK)],
                    upds[b], semm[b])
                pltpu.sync_copy(pos_hbm.at[pl.ds(base, SCHUNK)], poss[b])
                icp = pltpu.async_copy(dst_hbm.at[poss[b]], idxs[b], semi[b])
                return mcp, icp

            cps = {0: issue(0, 0)}
            for i in range(nchunk):
                cur, nxt = i % 2, (i + 1) % 2
                if i + 1 < nchunk:
                    cps[nxt] = issue(i + 1, nxt)
                for cp in cps[cur]:
                    cp.wait()
                pltpu.sync_copy(upds[cur], acc_sh.at[idxs[cur]], add=True)

    plsc.subcore_barrier()

    row0 = c * N_NODES + s * ROWS_PER_TILE
    pltpu.sync_copy(acc_sh.at[pl.ds(s * ROWS_PER_TILE, ROWS_PER_TILE)],
                    out_hbm.at[pl.ds(row0, ROWS_PER_TILE)])


def _scatter_add(mq, dst, pos):
    return pl.kernel(
        _scatter_body,
        out_type=jax.ShapeDtypeStruct((NC * N_NODES, 64), jnp.float32),
        mesh=plsc.VectorSubcoreMesh(core_axis_name="c", subcore_axis_name="s"),
        scratch_types=[
            pltpu.VMEM((SCHUNK,), jnp.int32),
            pltpu.VMEM((SCHUNK,), jnp.int32),
            pltpu.VMEM((SCHUNK,), jnp.int32),
            pltpu.VMEM((SCHUNK,), jnp.int32),
            pltpu.VMEM((SCHUNK, 64), jnp.float32),
            pltpu.VMEM((SCHUNK, 64), jnp.float32),
            pltpu.SemaphoreType.DMA,
            pltpu.SemaphoreType.DMA,
            pltpu.SemaphoreType.DMA,
            pltpu.SemaphoreType.DMA,
            pltpu.VMEM_SHARED((N_NODES, 64), jnp.float32),
        ],
        compiler_params=pltpu.CompilerParams(use_tc_tiling_on_sc=False),
    )(mq[0], mq[1], mq[2], mq[3], dst, pos)


def _post_body(a_ref, b_ref, x_ref, g_ref, bt_ref, w1_ref, b1_ref,
               w2_ref, b2_ref, w3_ref, b3_ref, o_ref):
    gnn = a_ref[...] + b_ref[...]
    pp = jnp.concatenate([gnn, x_ref[...]], axis=1)
    mean = jnp.mean(pp, axis=-1, keepdims=True)
    var = jnp.mean((pp - mean) ** 2, axis=-1, keepdims=True)
    h = (pp - mean) / jnp.sqrt(var + 1e-5) * g_ref[...] + bt_ref[...]
    h = _lrelu(jnp.dot(h, w1_ref[...],
                       preferred_element_type=jnp.float32) + b1_ref[...])
    h = _lrelu(jnp.dot(h, w2_ref[...],
                       preferred_element_type=jnp.float32) + b2_ref[...])
    o_ref[...] = jnp.dot(h, w3_ref[...],
                         preferred_element_type=jnp.float32) + b3_ref[...]


def _post(acc2, x, ln_gamma, ln_beta, pp_W1, pp_b1, pp_W2, pp_b2, pp_W3, pp_b3):
    BN = 1000
    nblk = N_NODES // BN
    full = lambda shape: pl.BlockSpec(shape, lambda i: (0, 0))
    return pl.pallas_call(
        _post_body,
        grid=(nblk,),
        in_specs=[
            pl.BlockSpec((BN, 64), lambda i: (i, 0)),
            pl.BlockSpec((BN, 64), lambda i: (i + N_NODES // BN, 0)),
            pl.BlockSpec((BN, 128), lambda i: (i, 0)),
            full((1, 192)), full((1, 192)),
            full((192, 32)), full((1, 32)), full((32, 32)), full((1, 32)),
            full((32, 128)), full((1, 128)),
        ],
        out_specs=pl.BlockSpec((BN, 128), lambda i: (i, 0)),
        out_shape=jax.ShapeDtypeStruct((N_NODES, 128), jnp.float32),
    )(acc2, acc2, x, ln_gamma.reshape(1, -1), ln_beta.reshape(1, -1),
      pp_W1, pp_b1.reshape(1, -1), pp_W2, pp_b2.reshape(1, -1),
      pp_W3, pp_b3.reshape(1, -1))


def kernel(x, edge_index, edge_attr, ne_W1, ne_b1, ne_W2, ne_b2,
           ee_W1, ee_b1, ee_W2, ee_b2, gn_W1, gn_b1, gn_W2, gn_b2,
           ln_gamma, ln_beta, pp_W1, pp_b1, pp_W2, pp_b2, pp_W3, pp_b3):
    src = edge_index[0].astype(jnp.int32)
    dst = edge_index[1].astype(jnp.int32)
    gnW1_e = gn_W1[:32]
    gnW1_n = gn_W1[32:]

    node_pre = _node_pre(x, ne_W1, ne_b1, ne_W2, ne_b2, gnW1_n)
    g = _gather(node_pre, src)
    g_p = jnp.reshape(g, (EP, 128))
    ea_p = jnp.reshape(edge_attr, (EP, 128))
    mq = _messages(ea_p, g_p, ee_W1, ee_b1, ee_W2, ee_b2, gnW1_e,
                   gn_b1, gn_W2, gn_b2)
    mq = [jnp.reshape(m, (EQ, 64)) for m in mq]
    q = jnp.arange(EQ, dtype=jnp.int32)
    qbase = 8 * (q // 2) + q % 2
    pos = jnp.concatenate([qbase + 2 * j for j in range(4)])
    acc2 = _scatter_add(mq, dst, pos)
    return _post(acc2, x, ln_gamma, ln_beta,
                 pp_W1, pp_b1, pp_W2, pp_b2, pp_W3, pp_b3)
